# speedup vs baseline: 2.9029x; 2.9029x over previous
import jax
import jax.numpy as jnp
from jax.experimental import pallas as pl

_BL = 512


def _add_body(x_ref, pos_ref, out_ref):
    out_ref[...] = x_ref[...] + pos_ref[...][None]


def kernel(x, pos_table):
    B, L, D = x.shape
    nl = L // _BL
    out = pl.pallas_call(
        _add_body,
        grid=(nl, B),
        in_specs=[
            pl.BlockSpec((1, _BL, D), lambda l, b: (b, l, 0)),
            pl.BlockSpec((_BL, D), lambda l, b: (l, 0)),
        ],
        out_specs=pl.BlockSpec((1, _BL, D), lambda l, b: (b, l, 0)),
        out_shape=jax.ShapeDtypeStruct((B, L, D), x.dtype),
    )(x, pos_table)
    return out
